# baseline (speedup 1.0000x reference)
import jax
import jax.numpy as jnp
from jax import lax
from jax.experimental import pallas as pl
from jax.experimental.pallas import tpu as pltpu
from jax.experimental.pallas import tpu_sc as plsc

SEQ_LEN = 2048
N_BATCH = 32
D_MODEL = 1024

NUM_WORKERS = 32
ROWS_PER_W = SEQ_LEN // NUM_WORKERS

R_TILE = 8
K_REP = 4
N_TILES = ROWS_PER_W // R_TILE
N_WGRP = N_BATCH // K_REP


def _sc_broadcast_body(pe_hbm, out_hbm, rep0, rep1, sem_in, sem_w0, sem_w1):
    c = lax.axis_index("c")
    s = lax.axis_index("s")
    wid = s * 2 + c
    base = wid * ROWS_PER_W

    reps = (rep0, rep1)
    wsems = (sem_w0, sem_w1)

    def stage(t, buf):
        src = pe_hbm.at[pl.ds(base + t * R_TILE, R_TILE)]
        for j in range(K_REP):
            pltpu.async_copy(src, buf.at[:, j], sem_in)

    stage(0, rep0)
    for t in range(N_TILES):
        buf = reps[t % 2]
        wsem = wsems[t % 2]
        for _ in range(K_REP):
            pltpu.make_async_copy(pe_hbm.at[pl.ds(base, R_TILE)],
                                  buf.at[:, 0], sem_in).wait()
        dst_rows = pl.ds(base + t * R_TILE, R_TILE)
        for m in range(N_WGRP):
            pltpu.async_copy(buf, out_hbm.at[dst_rows, pl.ds(m * K_REP, K_REP)],
                             wsem)
        if t + 1 < N_TILES:
            nxt = reps[(t + 1) % 2]
            if t >= 1:
                for _ in range(N_WGRP):
                    pltpu.make_async_copy(
                        nxt, out_hbm.at[pl.ds(base, R_TILE), pl.ds(0, K_REP)],
                        wsems[(t + 1) % 2]).wait()
            stage(t + 1, nxt)

    for i in range(2):
        for _ in range(N_WGRP):
            pltpu.make_async_copy(
                reps[i], out_hbm.at[pl.ds(base, R_TILE), pl.ds(0, K_REP)],
                wsems[i]).wait()


def kernel(z, pos_embed):
    del z
    mesh = plsc.VectorSubcoreMesh(core_axis_name="c", subcore_axis_name="s")
    return pl.kernel(
        _sc_broadcast_body,
        out_type=jax.ShapeDtypeStruct((SEQ_LEN, N_BATCH, D_MODEL), jnp.float32),
        mesh=mesh,
        scratch_types=[
            pltpu.VMEM((R_TILE, K_REP, D_MODEL), jnp.float32),
            pltpu.VMEM((R_TILE, K_REP, D_MODEL), jnp.float32),
            pltpu.SemaphoreType.DMA,
            pltpu.SemaphoreType.DMA,
            pltpu.SemaphoreType.DMA,
        ],
    )(pos_embed)


# speedup vs baseline: 1.2149x; 1.2149x over previous
import jax
import jax.numpy as jnp
from jax import lax
from jax.experimental import pallas as pl
from jax.experimental.pallas import tpu as pltpu
from jax.experimental.pallas import tpu_sc as plsc

SEQ_LEN = 2048
N_BATCH = 32
D_MODEL = 1024

NUM_WORKERS = 32
ROWS_PER_W = SEQ_LEN // NUM_WORKERS


def _sc_broadcast_body(pe_hbm, out_hbm, rows_v, sem_in, sem_out):
    c = lax.axis_index("c")
    s = lax.axis_index("s")
    wid = s * 2 + c
    base = wid * ROWS_PER_W

    pltpu.async_copy(pe_hbm.at[pl.ds(base, ROWS_PER_W)], rows_v, sem_in).wait()

    for n in range(N_BATCH):
        pltpu.async_copy(rows_v, out_hbm.at[pl.ds(base, ROWS_PER_W), n], sem_out)
    for _ in range(N_BATCH):
        pltpu.make_async_copy(rows_v, out_hbm.at[pl.ds(base, ROWS_PER_W), 0],
                              sem_out).wait()


def kernel(z, pos_embed):
    del z
    mesh = plsc.VectorSubcoreMesh(core_axis_name="c", subcore_axis_name="s")
    return pl.kernel(
        _sc_broadcast_body,
        out_type=jax.ShapeDtypeStruct((SEQ_LEN, N_BATCH, D_MODEL), jnp.float32),
        mesh=mesh,
        scratch_types=[
            pltpu.VMEM((ROWS_PER_W, D_MODEL), jnp.float32),
            pltpu.SemaphoreType.DMA,
            pltpu.SemaphoreType.DMA,
        ],
    )(pos_embed)
